# baseline (speedup 1.0000x reference)
import functools

import jax
import jax.numpy as jnp
from jax import lax
from jax.experimental import pallas as pl
from jax.experimental.pallas import tpu as pltpu
from jax.experimental.pallas import tpu_sc as plsc

N = 10000
E = 320000
D = 128
H = 64
NF = 128

NC = 2
NS = 16
NW = NC * NS
EW = E // NW
B = 1250
CH = EW // B
NP = 10240
RPT = NP // NS


def _sc_aggregate(p, row3d, col3d, zeros_nh):
    mesh = plsc.VectorSubcoreMesh(core_axis_name="c", subcore_axis_name="s")

    @functools.partial(
        pl.kernel,
        out_type=jax.ShapeDtypeStruct((NC, NP, H), jnp.bfloat16),
        mesh=mesh,
        scratch_types=[
            pltpu.VMEM_SHARED((NP, H), jnp.bfloat16),
            pltpu.VMEM((CH, B), jnp.int32),
            pltpu.VMEM((CH, B), jnp.int32),
            pltpu.VMEM((B, H), jnp.bfloat16),
            pltpu.VMEM((B, H), jnp.bfloat16),
            pltpu.SemaphoreType.DMA,
            pltpu.SemaphoreType.DMA,
        ],
        compiler_params=pltpu.CompilerParams(use_tc_tiling_on_sc=False),
    )
    def k(p_hbm, row_hbm, col_hbm, z_hbm, out_hbm, acc, colc, rowc,
          g0, g1, s0, s1):
        c = lax.axis_index("c")
        s = lax.axis_index("s")
        w = s * NC + c
        r0 = s * RPT
        pltpu.sync_copy(z_hbm.at[pl.ds(r0, RPT)], acc.at[pl.ds(r0, RPT)])
        pltpu.sync_copy(col_hbm.at[w], colc)
        pltpu.sync_copy(row_hbm.at[w], rowc)
        plsc.subcore_barrier()

        pltpu.async_copy(p_hbm.at[colc.at[0]], g0, s0)

        def body(i, carry):
            j0 = 2 * i
            pltpu.async_copy(p_hbm.at[colc.at[j0 + 1]], g1, s1)
            pltpu.make_async_copy(p_hbm.at[colc.at[j0]], g0, s0).wait()
            pltpu.sync_copy(g0, acc.at[rowc.at[j0]], add=True)

            @pl.when(j0 + 2 < CH)
            def _():
                pltpu.async_copy(p_hbm.at[colc.at[j0 + 2]], g0, s0)

            pltpu.make_async_copy(p_hbm.at[colc.at[j0 + 1]], g1, s1).wait()
            pltpu.sync_copy(g1, acc.at[rowc.at[j0 + 1]], add=True)
            return carry

        lax.fori_loop(0, CH // 2, body, 0)
        plsc.subcore_barrier()
        pltpu.sync_copy(acc.at[pl.ds(r0, RPT)], out_hbm.at[c, pl.ds(r0, RPT)])

    return k(p, row3d, col3d, zeros_nh)


def _mm_first_body(x_ref, w_ref, o_ref):
    o_ref[...] = lax.dot_general(
        x_ref[...], w_ref[...], (((1,), (1,)), ((), ())),
        preferred_element_type=jnp.float32).astype(jnp.bfloat16)


def _mm_first(x, w):
    return pl.pallas_call(
        _mm_first_body,
        out_shape=jax.ShapeDtypeStruct((NP, H), jnp.bfloat16),
    )(x, w)


def _mm_layer_body(a_ref, b_ref, w_ref, o_ref):
    asum = a_ref[0].astype(jnp.float32) + a_ref[1].astype(jnp.float32)
    h = jnp.maximum(asum + b_ref[...], 0.0)
    o_ref[...] = lax.dot_general(
        h, w_ref[...], (((1,), (1,)), ((), ())),
        preferred_element_type=jnp.float32).astype(jnp.bfloat16)


def _mm_layer(a, b, w):
    return pl.pallas_call(
        _mm_layer_body,
        out_shape=jax.ShapeDtypeStruct((NP, H), jnp.bfloat16),
    )(a, b.reshape(1, H), w)


def _head_body(a_ref, b_ref, f_ref, wf1_ref, bf1_ref, wf2_ref, bf2_ref, o_ref):
    asum = (a_ref[0, :N].astype(jnp.float32)
            + a_ref[1, :N].astype(jnp.float32))
    h = jnp.maximum(asum + b_ref[...], 0.0)
    g = jnp.sum(h, axis=0, keepdims=True) * (1.0 / N)
    g = jnp.concatenate([g, f_ref[...]], axis=1)
    z = lax.dot_general(g, wf1_ref[...], (((1,), (1,)), ((), ())),
                        preferred_element_type=jnp.float32)
    z = jnp.maximum(z + bf1_ref[...], 0.0)
    t = jnp.sum(z * wf2_ref[...])
    o_ref[...] = jax.nn.sigmoid(t + bf2_ref[...])


def _head(a, b3, frag, wf1, bf1, wf2, bf2):
    out = pl.pallas_call(
        _head_body,
        out_shape=jax.ShapeDtypeStruct((1, 128), jnp.float32),
    )(a, b3.reshape(1, H), frag.reshape(1, NF), wf1, bf1.reshape(1, -1),
      wf2, jnp.broadcast_to(bf2.reshape(1, 1), (1, 128)))
    return out[0, :1]


def kernel(x, edge_index, fragment_features, W1, b1, W2, b2, W3, b3,
           Wf1, bf1, Wf2, bf2):
    row3d = edge_index[0].astype(jnp.int32).reshape(NW, CH, B)
    col3d = edge_index[1].astype(jnp.int32).reshape(NW, CH, B)
    zeros_nh = jnp.zeros((NP, H), jnp.bfloat16)
    x_pad = jnp.pad(x, ((0, NP - N), (0, 0)))

    p = _mm_first(x_pad, W1)
    a = _sc_aggregate(p, row3d, col3d, zeros_nh)
    p = _mm_layer(a, b1, W2)
    a = _sc_aggregate(p, row3d, col3d, zeros_nh)
    p = _mm_layer(a, b2, W3)
    a = _sc_aggregate(p, row3d, col3d, zeros_nh)
    return _head(a, b3, fragment_features, Wf1, bf1, Wf2, bf2)


# speedup vs baseline: 1.0460x; 1.0460x over previous
import functools

import jax
import jax.numpy as jnp
from jax import lax
from jax.experimental import pallas as pl
from jax.experimental.pallas import tpu as pltpu
from jax.experimental.pallas import tpu_sc as plsc

N = 10000
E = 320000
D = 128
H = 64
NF = 128

NC = 2
NS = 16
NW = NC * NS
EW = E // NW
B = 1000
CH = EW // B
NP = 10240
RPT = NP // NS


def _sc_aggregate(p, row_f, col_f, zeros_nh):
    mesh = plsc.VectorSubcoreMesh(core_axis_name="c", subcore_axis_name="s")

    @functools.partial(
        pl.kernel,
        out_type=jax.ShapeDtypeStruct((NC, NP, H), jnp.bfloat16),
        mesh=mesh,
        scratch_types=[
            pltpu.VMEM_SHARED((NP, H), jnp.bfloat16),
            pltpu.VMEM((EW,), jnp.int32),
            pltpu.VMEM((EW,), jnp.int32),
            pltpu.VMEM((B, H), jnp.bfloat16),
            pltpu.VMEM((B, H), jnp.bfloat16),
            pltpu.SemaphoreType.DMA,
            pltpu.SemaphoreType.DMA,
        ],
        compiler_params=pltpu.CompilerParams(use_tc_tiling_on_sc=False),
    )
    def k(p_hbm, row_hbm, col_hbm, z_hbm, out_hbm, acc, colc, rowc,
          g0, g1, s0, s1):
        c = lax.axis_index("c")
        s = lax.axis_index("s")
        w = s * NC + c
        r0 = s * RPT
        pltpu.sync_copy(z_hbm.at[pl.ds(r0, RPT)], acc.at[pl.ds(r0, RPT)])
        pltpu.sync_copy(col_hbm.at[pl.ds(w * EW, EW)], colc)
        pltpu.sync_copy(row_hbm.at[pl.ds(w * EW, EW)], rowc)
        plsc.subcore_barrier()

        pltpu.async_copy(p_hbm.at[colc.at[pl.ds(0, B)]], g0, s0)

        def body(i, carry):
            j0 = 2 * i * B
            pltpu.async_copy(p_hbm.at[colc.at[pl.ds(j0 + B, B)]], g1, s1)
            pltpu.make_async_copy(
                p_hbm.at[colc.at[pl.ds(j0, B)]], g0, s0).wait()
            pltpu.sync_copy(g0, acc.at[rowc.at[pl.ds(j0, B)]], add=True)

            @pl.when(j0 + 2 * B < EW)
            def _():
                pltpu.async_copy(p_hbm.at[colc.at[pl.ds(j0 + 2 * B, B)]],
                                 g0, s0)

            pltpu.make_async_copy(
                p_hbm.at[colc.at[pl.ds(j0 + B, B)]], g1, s1).wait()
            pltpu.sync_copy(g1, acc.at[rowc.at[pl.ds(j0 + B, B)]], add=True)
            return carry

        lax.fori_loop(0, CH // 2, body, 0)
        plsc.subcore_barrier()
        pltpu.sync_copy(acc.at[pl.ds(r0, RPT)], out_hbm.at[c, pl.ds(r0, RPT)])

    return k(p, row_f, col_f, zeros_nh)


def _mm_first_body(x_ref, w_ref, o_ref):
    o_ref[...] = lax.dot_general(
        x_ref[...], w_ref[...], (((1,), (1,)), ((), ())),
        preferred_element_type=jnp.float32).astype(jnp.bfloat16)


def _mm_first(x, w):
    return pl.pallas_call(
        _mm_first_body,
        out_shape=jax.ShapeDtypeStruct((NP, H), jnp.bfloat16),
    )(x, w)


def _mm_layer_body(a_ref, b_ref, w_ref, o_ref):
    asum = a_ref[0].astype(jnp.float32) + a_ref[1].astype(jnp.float32)
    h = jnp.maximum(asum + b_ref[...], 0.0)
    o_ref[...] = lax.dot_general(
        h, w_ref[...], (((1,), (1,)), ((), ())),
        preferred_element_type=jnp.float32).astype(jnp.bfloat16)


def _mm_layer(a, b, w):
    return pl.pallas_call(
        _mm_layer_body,
        out_shape=jax.ShapeDtypeStruct((NP, H), jnp.bfloat16),
    )(a, b.reshape(1, H), w)


def _head_body(a_ref, b_ref, f_ref, wf1_ref, bf1_ref, wf2_ref, bf2_ref, o_ref):
    asum = (a_ref[0, :N].astype(jnp.float32)
            + a_ref[1, :N].astype(jnp.float32))
    h = jnp.maximum(asum + b_ref[...], 0.0)
    g = jnp.sum(h, axis=0, keepdims=True) * (1.0 / N)
    g = jnp.concatenate([g, f_ref[...]], axis=1)
    z = lax.dot_general(g, wf1_ref[...], (((1,), (1,)), ((), ())),
                        preferred_element_type=jnp.float32)
    z = jnp.maximum(z + bf1_ref[...], 0.0)
    t = jnp.sum(z * wf2_ref[...])
    o_ref[...] = jax.nn.sigmoid(t + bf2_ref[...])


def _head(a, b3, frag, wf1, bf1, wf2, bf2):
    out = pl.pallas_call(
        _head_body,
        out_shape=jax.ShapeDtypeStruct((1, 128), jnp.float32),
    )(a, b3.reshape(1, H), frag.reshape(1, NF), wf1, bf1.reshape(1, -1),
      wf2, jnp.broadcast_to(bf2.reshape(1, 1), (1, 128)))
    return out[0, :1]


def kernel(x, edge_index, fragment_features, W1, b1, W2, b2, W3, b3,
           Wf1, bf1, Wf2, bf2):
    row_f = edge_index[0].astype(jnp.int32)
    col_f = edge_index[1].astype(jnp.int32)
    zeros_nh = jnp.zeros((NP, H), jnp.bfloat16)
    x_pad = jnp.pad(x, ((0, NP - N), (0, 0)))

    p = _mm_first(x_pad, W1)
    a = _sc_aggregate(p, row_f, col_f, zeros_nh)
    p = _mm_layer(a, b1, W2)
    a = _sc_aggregate(p, row_f, col_f, zeros_nh)
    p = _mm_layer(a, b2, W3)
    a = _sc_aggregate(p, row_f, col_f, zeros_nh)
    return _head(a, b3, fragment_features, Wf1, bf1, Wf2, bf2)
